# baseline (speedup 1.0000x reference)
import jax
import jax.numpy as jnp
from jax.experimental import pallas as pl

_TOP_K = 2
_NUM_EXPERTS = 8
_BLOCK_T = 2048


def _top2(logits, i1_ref, i2_ref, g1_ref, g2_ref):
    lt = logits.T
    e8 = jax.lax.broadcasted_iota(jnp.int32, lt.shape, 0)
    m1 = jnp.max(lt, axis=0, keepdims=True)
    i1 = jnp.min(jnp.where(lt == m1, e8, _NUM_EXPERTS), axis=0, keepdims=True)
    masked = jnp.where(e8 == i1, -jnp.inf, lt)
    m2 = jnp.max(masked, axis=0, keepdims=True)
    i2 = jnp.min(jnp.where(masked == m2, e8, _NUM_EXPERTS), axis=0, keepdims=True)
    ex = jnp.exp(m2 - m1)
    den = 1.0 + ex
    bt = lt.shape[1]
    i1_ref[...] = i1.reshape(1, 1, bt)
    i2_ref[...] = i2.reshape(1, 1, bt)
    g1_ref[...] = (1.0 / den).reshape(1, 1, bt)
    g2_ref[...] = (ex / den).reshape(1, 1, bt)


def _router_block(xa_ref, xb_ref, w_ref,
                  i1a_ref, i2a_ref, g1a_ref, g2a_ref,
                  i1b_ref, i2b_ref, g1b_ref, g2b_ref):
    dn = (((1,), (0,)), ((), ()))
    la = jax.lax.dot_general(xa_ref[...], w_ref[...], dimension_numbers=dn,
                             preferred_element_type=jnp.float32)
    _top2(la, i1a_ref, i2a_ref, g1a_ref, g2a_ref)
    lb = jax.lax.dot_general(xb_ref[...], w_ref[...], dimension_numbers=dn,
                             preferred_element_type=jnp.float32)
    _top2(lb, i1b_ref, i2b_ref, g1b_ref, g2b_ref)


@jax.jit
def kernel(x, W):
    n_tokens, d_model = x.shape
    nb = n_tokens // (2 * _BLOCK_T)
    wt = W.T
    row_spec = pl.BlockSpec((1, 1, _BLOCK_T), lambda i: (i, 0, 0))
    row_i = jax.ShapeDtypeStruct((nb, 1, _BLOCK_T), jnp.int32)
    row_f = jax.ShapeDtypeStruct((nb, 1, _BLOCK_T), jnp.float32)
    outs = pl.pallas_call(
        _router_block,
        grid=(nb,),
        in_specs=[
            pl.BlockSpec((_BLOCK_T, d_model), lambda i: (2 * i, 0)),
            pl.BlockSpec((_BLOCK_T, d_model), lambda i: (2 * i + 1, 0)),
            pl.BlockSpec((d_model, _NUM_EXPERTS), lambda i: (0, 0)),
        ],
        out_specs=[row_spec] * 8,
        out_shape=[row_i, row_i, row_f, row_f, row_i, row_i, row_f, row_f],
    )(x, x, wt)
    i1a, i2a, g1a, g2a, i1b, i2b, g1b, g2b = outs

    def weave(a, b):
        return jnp.stack([a.reshape(nb, _BLOCK_T), b.reshape(nb, _BLOCK_T)],
                         axis=1).reshape(-1)

    idx = jnp.stack([weave(i1a, i1b), weave(i2a, i2b)], axis=1)
    gates = jnp.stack([weave(g1a, g1b), weave(g2a, g2b)], axis=1)
    return idx, gates
